# Initial kernel scaffold; baseline (speedup 1.0000x reference)
import jax
import jax.numpy as jnp
from jax.experimental import pallas as pl


def kernel(hidden_states, W, b):
    raise NotImplementedError("write your pallas kernel here")


# speedup vs baseline: 1.3130x; 1.3130x over previous
import functools
import math

import jax
import jax.numpy as jnp
from jax.experimental import pallas as pl
from jax.experimental.pallas import tpu as pltpu

KEEP = 0.5
_I32_MIN = -(2 ** 31)


def _order_key(x):
    b = jax.lax.bitcast_convert_type(x, jnp.int32)
    return jnp.where(b >= 0, b, jnp.bitwise_xor(jnp.invert(b), jnp.int32(_I32_MIN)))


def _scores_kernel(h_ref, w_ref, b_ref, s_ref):
    h = h_ref[0].astype(jnp.bfloat16).astype(jnp.float32)
    w = w_ref[0].astype(jnp.bfloat16).astype(jnp.float32)
    s = jnp.sum(h * w, axis=1, keepdims=True)
    s_ref[0] = s + b_ref[0, 0]


def _mask_kernel(k, s_wide_ref, s_col_ref, h_ref, p_ref, m_ref, thr_ref):
    t = pl.program_id(1)

    @pl.when(t == 0)
    def _():
        keys = _order_key(s_wide_ref[0])

        def body(i, prefix_u):
            j = 31 - i
            cand_u = jnp.bitwise_or(prefix_u, jnp.left_shift(jnp.int32(1), j))
            cand_i = jnp.bitwise_xor(cand_u, jnp.int32(_I32_MIN))
            cnt = jnp.sum((keys >= cand_i).astype(jnp.int32))
            return jnp.where(cnt >= k, cand_u, prefix_u)

        prefix_u = jax.lax.fori_loop(0, 32, body, jnp.int32(0))
        thr_ref[0] = jnp.bitwise_xor(prefix_u, jnp.int32(_I32_MIN))

    keys_blk = _order_key(s_col_ref[0])
    keep = keys_blk >= thr_ref[0]
    m_ref[0] = keep
    p_ref[0] = h_ref[0] * keep.astype(jnp.float32)


def _build(B, T, D, k, interpret=False):
    BT1 = 1024
    scores_call = pl.pallas_call(
        _scores_kernel,
        grid=(B, T // BT1),
        in_specs=[
            pl.BlockSpec((1, BT1, D), lambda i, j: (i, j, 0)),
            pl.BlockSpec((1, D), lambda i, j: (0, 0)),
            pl.BlockSpec((1, 1), lambda i, j: (0, 0)),
        ],
        out_specs=pl.BlockSpec((1, BT1, 1), lambda i, j: (i, j, 0)),
        out_shape=jax.ShapeDtypeStruct((B, T, 1), jnp.float32),
        interpret=interpret,
    )

    BT2 = 512
    mask_call = pl.pallas_call(
        functools.partial(_mask_kernel, k),
        grid=(B, T // BT2),
        in_specs=[
            pl.BlockSpec((1, T // 128, 128), lambda i, j: (i, 0, 0)),
            pl.BlockSpec((1, BT2, 1), lambda i, j: (i, j, 0)),
            pl.BlockSpec((1, BT2, D), lambda i, j: (i, j, 0)),
        ],
        out_specs=[
            pl.BlockSpec((1, BT2, D), lambda i, j: (i, j, 0)),
            pl.BlockSpec((1, BT2, 1), lambda i, j: (i, j, 0)),
        ],
        out_shape=[
            jax.ShapeDtypeStruct((B, T, D), jnp.float32),
            jax.ShapeDtypeStruct((B, T, 1), jnp.bool_),
        ],
        scratch_shapes=[pltpu.SMEM((1,), jnp.int32)],
        interpret=interpret,
    )
    return scores_call, mask_call


def kernel(hidden_states, W, b, interpret=False):
    B, T, D = hidden_states.shape
    k = min(max(1, math.ceil(KEEP * T)), T)
    scores_call, mask_call = _build(B, T, D, k, interpret)
    scores_col = scores_call(hidden_states, W, b.reshape(1, 1))
    scores_wide = scores_col.reshape(B, T // 128, 128)
    pruned, mask_col = mask_call(scores_wide, scores_col, hidden_states)
    return (pruned, mask_col.reshape(B, T), scores_col.reshape(B, T))


# speedup vs baseline: 1.4374x; 1.0947x over previous
import functools
import math

import jax
import jax.numpy as jnp
from jax.experimental import pallas as pl
from jax.experimental.pallas import tpu as pltpu

KEEP = 0.5
_I32_MIN = -(2 ** 31)


def _order_key(x):
    b = jax.lax.bitcast_convert_type(x, jnp.int32)
    return jnp.where(b >= 0, b, jnp.bitwise_xor(jnp.invert(b), jnp.int32(_I32_MIN)))


def _fused_kernel(k, nt, h_ref, w_ref, b_ref, p_ref, m_ref, s_ref,
                  hrow_ref, srow_ref, swide_ref, thr_ref):
    p = pl.program_id(1)
    t = pl.program_id(2)
    bt = h_ref.shape[1]

    @pl.when(p == 0)
    def _phase0():
        h_raw = h_ref[0]
        hrow_ref[pl.ds(t * bt, bt), :] = h_raw
        h = h_raw.astype(jnp.bfloat16).astype(jnp.float32)
        w = w_ref[0].astype(jnp.bfloat16).astype(jnp.float32)
        s = jnp.sum(h * w, axis=1, keepdims=True) + b_ref[0, 0]
        srow_ref[pl.ds(t * bt, bt), :] = s
        swide_ref[pl.ds(t * (bt // 128), bt // 128), :] = s.reshape(bt // 128, 128)
        s_ref[0] = s

        @pl.when(t == nt - 1)
        def _():
            keys = _order_key(swide_ref[...])

            def body(i, prefix_u):
                j = 31 - i
                cand_u = jnp.bitwise_or(prefix_u, jnp.left_shift(jnp.int32(1), j))
                cand_i = jnp.bitwise_xor(cand_u, jnp.int32(_I32_MIN))
                cnt = jnp.sum((keys >= cand_i).astype(jnp.int32))
                return jnp.where(cnt >= k, cand_u, prefix_u)

            prefix_u = jax.lax.fori_loop(0, 32, body, jnp.int32(0))
            thr_ref[0] = jnp.bitwise_xor(prefix_u, jnp.int32(_I32_MIN))

    @pl.when(p == 1)
    def _phase1():
        s_tile = srow_ref[pl.ds(t * bt, bt), :]
        keep = _order_key(s_tile) >= thr_ref[0]
        m_ref[0] = keep
        p_ref[0] = hrow_ref[pl.ds(t * bt, bt), :] * keep.astype(jnp.float32)
        s_ref[0] = s_tile


def _run(hidden_states, W, b, interpret=False):
    B, T, D = hidden_states.shape
    k = min(max(1, math.ceil(KEEP * T)), T)
    BT = 512
    nt = T // BT
    pruned, mask_col, scores_col = pl.pallas_call(
        functools.partial(_fused_kernel, k, nt),
        grid=(B, 2, nt),
        in_specs=[
            pl.BlockSpec((1, BT, D), lambda i, p, j: (i, j * (1 - p), 0)),
            pl.BlockSpec((1, D), lambda i, p, j: (0, 0)),
            pl.BlockSpec((1, 1), lambda i, p, j: (0, 0)),
        ],
        out_specs=[
            pl.BlockSpec((1, BT, D), lambda i, p, j: (i, j * p, 0)),
            pl.BlockSpec((1, BT, 1), lambda i, p, j: (i, j * p, 0)),
            pl.BlockSpec((1, BT, 1), lambda i, p, j: (i, j, 0)),
        ],
        out_shape=[
            jax.ShapeDtypeStruct((B, T, D), jnp.float32),
            jax.ShapeDtypeStruct((B, T, 1), jnp.bool_),
            jax.ShapeDtypeStruct((B, T, 1), jnp.float32),
        ],
        scratch_shapes=[
            pltpu.VMEM((T, D), jnp.float32),
            pltpu.VMEM((T, 1), jnp.float32),
            pltpu.VMEM((T // 128, 128), jnp.float32),
            pltpu.SMEM((1,), jnp.int32),
        ],
        interpret=interpret,
    )(hidden_states, W, b.reshape(1, 1))
    return (pruned, mask_col.reshape(B, T), scores_col.reshape(B, T))


def kernel(hidden_states, W, b, interpret=False):
    return _run(hidden_states, W, b, interpret)
